# Initial kernel scaffold; baseline (speedup 1.0000x reference)
import jax
import jax.numpy as jnp
from jax.experimental import pallas as pl


def kernel(hidden_states, router_w, up_w, down_w):
    raise NotImplementedError("write your pallas kernel here")


# speedup vs baseline: 1.6241x; 1.6241x over previous
import functools

import jax
import jax.numpy as jnp
from jax.experimental import pallas as pl
from jax.experimental.pallas import tpu as pltpu
from jax.experimental.pallas import tpu_sc as plsc

N = 2048
H = 2048
FF = 4096
E = 8
K = 2
BM = 512
BF = 512
NFF = FF // BF
NPAD = N * K + E * BM
NBLK = NPAD // BM
GW = 128
SPLIT = 8


def _router_body(x_ref, rw_ref, w_ref, i_ref, aux_ref):
    x = x_ref[...]
    rw = rw_ref[...]
    logits = jax.lax.dot_general(
        x, rw, (((1,), (1,)), ((), ())), preferred_element_type=jnp.float32)
    m = jnp.max(logits, axis=1, keepdims=True)
    ex = jnp.exp(logits - m)
    probs = ex / jnp.sum(ex, axis=1, keepdims=True)

    ids = jax.lax.broadcasted_iota(jnp.int32, (N, E), 1)
    max1 = jnp.max(probs, axis=1, keepdims=True)
    idx1 = jnp.min(jnp.where(probs == max1, ids, E), axis=1, keepdims=True)
    masked = jnp.where(ids == idx1, -jnp.inf, probs)
    max2 = jnp.max(masked, axis=1, keepdims=True)
    idx2 = jnp.min(jnp.where(masked == max2, ids, E), axis=1, keepdims=True)

    denom = max1 + max2
    w_ref[...] = jnp.concatenate([max1 / denom, max2 / denom], axis=1)
    i_ref[...] = jnp.concatenate([idx1, idx2], axis=1).astype(jnp.int32)

    one_hot = ((ids == idx1) | (ids == idx2)).astype(jnp.float32)
    f = jnp.sum(one_hot, axis=0, keepdims=True) / (N * K)
    p_mean = jnp.mean(probs, axis=0, keepdims=True)
    aux_ref[...] = E * jnp.sum(f * p_mean, axis=1, keepdims=True)


def _router(flat, router_w):
    return pl.pallas_call(
        _router_body,
        out_shape=(
            jax.ShapeDtypeStruct((N, K), jnp.float32),
            jax.ShapeDtypeStruct((N, K), jnp.int32),
            jax.ShapeDtypeStruct((1, 1), jnp.float32),
        ),
    )(flat, router_w)


def _sc_gather_rows(src, idx, out_rows):
    cols = src.shape[1]
    sub_cols = cols // SPLIT
    n_sub = out_rows * SPLIT
    src8 = src.reshape(src.shape[0] * SPLIT, sub_cols)
    idx8 = (idx[:, None] * SPLIT
            + jnp.arange(SPLIT, dtype=jnp.int32)[None, :]).reshape(1, n_sub)
    mesh = plsc.VectorSubcoreMesh(core_axis_name="c", subcore_axis_name="s")

    @functools.partial(
        pl.kernel,
        out_type=jax.ShapeDtypeStruct((n_sub, sub_cols), src.dtype),
        mesh=mesh)
    def k(x_hbm, i_hbm, o_hbm):
        def body(i_vmem, o_vmem):
            pltpu.sync_copy(x_hbm.at[i_vmem.at[0]], o_vmem)

        pltpu.emit_pipeline(
            body,
            grid=(n_sub // GW,),
            in_specs=[pl.BlockSpec((1, GW), lambda i: (0, i))],
            out_specs=[pl.BlockSpec((GW, sub_cols), lambda i: (i, 0))],
            core_axis_name=("c", "s"),
            dimension_semantics=(pltpu.PARALLEL,),
        )(i_hbm, o_hbm)

    return k(src8, idx8).reshape(out_rows, cols)


def _ffn_body(be_ref, ip_ref, jm_ref, bv_ref, x_ref, up_ref, dn_ref, o_ref):
    i = pl.program_id(0)
    j = pl.program_id(1)

    @pl.when(bv_ref[i] == 1)
    def _():
        xb = x_ref[...].astype(jnp.bfloat16)
        up = up_ref[0].astype(jnp.bfloat16)
        h = jax.lax.dot_general(
            xb, up, (((1,), (0,)), ((), ())), preferred_element_type=jnp.float32)
        h = 0.5 * h * (1.0 + jax.lax.erf(h * 0.7071067811865476))
        dn = dn_ref[0].astype(jnp.bfloat16)
        acc = jax.lax.dot_general(
            h.astype(jnp.bfloat16), dn, (((1,), (0,)), ((), ())),
            preferred_element_type=jnp.float32)

        @pl.when(j == 0)
        def _():
            o_ref[...] = acc

        @pl.when(j > 0)
        def _():
            o_ref[...] += acc


def _ffn(xs, up_w, down_w, be, ip, jm, bv):
    grid_spec = pltpu.PrefetchScalarGridSpec(
        num_scalar_prefetch=4,
        grid=(NBLK, NFF),
        in_specs=[
            pl.BlockSpec((BM, H), lambda i, j, be, ip, jm, bv: (ip[i], 0)),
            pl.BlockSpec((1, H, BF), lambda i, j, be, ip, jm, bv: (be[i], 0, jm[i, j])),
            pl.BlockSpec((1, BF, H), lambda i, j, be, ip, jm, bv: (be[i], jm[i, j], 0)),
        ],
        out_specs=pl.BlockSpec((BM, H), lambda i, j, be, ip, jm, bv: (i, 0)),
    )
    return pl.pallas_call(
        _ffn_body,
        grid_spec=grid_spec,
        out_shape=jax.ShapeDtypeStruct((NPAD, H), jnp.float32),
        compiler_params=pltpu.CompilerParams(
            dimension_semantics=("arbitrary", "arbitrary")),
    )(be, ip, jm, bv, xs, up_w, down_w)


def _combine_body(g0_ref, g1_ref, w_ref, o_ref):
    w = w_ref[...]
    o_ref[...] = w[:, 0:1] * g0_ref[...] + w[:, 1:2] * g1_ref[...]


def _combine(g0, g1, top_w):
    bt = 256
    return pl.pallas_call(
        _combine_body,
        out_shape=jax.ShapeDtypeStruct((N, H), jnp.float32),
        grid=(N // bt,),
        in_specs=[
            pl.BlockSpec((bt, H), lambda i: (i, 0)),
            pl.BlockSpec((bt, H), lambda i: (i, 0)),
            pl.BlockSpec((bt, K), lambda i: (i, 0)),
        ],
        out_specs=pl.BlockSpec((bt, H), lambda i: (i, 0)),
    )(g0, g1, top_w)


def kernel(hidden_states, router_w, up_w, down_w):
    b, s, h = hidden_states.shape
    flat = hidden_states.reshape(N, H)

    top_w, top_idx, aux = _router(flat, router_w)

    eid = top_idx.reshape(-1)
    onehot = (eid[:, None] == jnp.arange(E)[None, :]).astype(jnp.int32)
    cum = jnp.cumsum(onehot, axis=0)
    counts = cum[-1]
    rank = jnp.take_along_axis(cum, eid[:, None], axis=1)[:, 0] - 1
    pc = ((counts + BM - 1) // BM) * BM
    pad_start = jnp.concatenate(
        [jnp.zeros((1,), counts.dtype), jnp.cumsum(pc)])[:E]
    pos = pad_start[eid] + rank
    blk_cum = jnp.cumsum(pc // BM)
    total_blocks = blk_cum[-1]
    bids = jnp.arange(NBLK, dtype=jnp.int32)
    block_expert = jnp.minimum(
        jnp.searchsorted(blk_cum, bids, side="right"), E - 1).astype(jnp.int32)
    block_valid = (bids < total_blocks).astype(jnp.int32)
    jj = jnp.arange(NFF, dtype=jnp.int32)
    serp = jnp.where((bids[:, None] % 2) == 0, jj[None, :], NFF - 1 - jj[None, :])
    jpin = jnp.where(((total_blocks - 1) % 2) == 0, NFF - 1, 0)
    jm = jnp.where(block_valid[:, None] == 1, serp, jpin).astype(jnp.int32)
    ip = jnp.where(block_valid == 1, bids, total_blocks - 1).astype(jnp.int32)

    inv = jnp.zeros((NPAD,), jnp.int32).at[pos].set(
        jnp.arange(N * K, dtype=jnp.int32) // K)

    xs = _sc_gather_rows(flat, inv, NPAD)
    ffn_out = _ffn(xs, up_w, down_w, block_expert, ip, jm, block_valid)

    pos_r = pos.reshape(N, K)
    gath_idx = jnp.concatenate([pos_r[:, 0], pos_r[:, 1]])
    g = _sc_gather_rows(ffn_out, gath_idx, N * K)
    out = _combine(g[:N], g[N:], top_w)

    return out.reshape(b, s, h), aux.reshape(())
